# baseline (speedup 1.0000x reference)
import functools

import jax
import jax.numpy as jnp
from jax import lax
from jax.experimental import pallas as pl
from jax.experimental.pallas import tpu as pltpu
from jax.experimental.pallas import tpu_sc as plsc

N_NODES = 10000
D_FEAT = 256
L = 16
NC, NS = 2, 16
NW = NC * NS
K = 64
SLOW_CORE = 1
T_SLOW = 3200
T_FAST = 7040


@functools.cache
def _build(E_pad):
    assert E_pad == NS * (T_SLOW + T_FAST)
    mesh = plsc.VectorSubcoreMesh(core_axis_name="c", subcore_axis_name="s")

    @functools.partial(
        pl.kernel,
        mesh=mesh,
        out_type=jax.ShapeDtypeStruct((E_pad,), jnp.float32),
        scratch_types=[
            pltpu.VMEM((T_FAST,), jnp.int32),
            pltpu.VMEM((T_FAST,), jnp.int32),
            pltpu.VMEM((K, D_FEAT), jnp.float32),
            pltpu.VMEM((K, D_FEAT), jnp.float32),
            pltpu.VMEM((K, D_FEAT), jnp.float32),
            pltpu.VMEM((K, D_FEAT), jnp.float32),
            pltpu.VMEM((T_FAST,), jnp.float32),
            pltpu.SemaphoreType.DMA,
            pltpu.SemaphoreType.DMA,
            pltpu.SemaphoreType.DMA,
            pltpu.SemaphoreType.DMA,
        ],
    )
    def edge_dot(h_hbm, src_hbm, dst_hbm, out_hbm,
                 idx_s, idx_d, rows_s0, rows_s1, rows_d0, rows_d1,
                 out_v, sem_s0, sem_s1, sem_d0, sem_d1):
        c = lax.axis_index("c")
        s = lax.axis_index("s")
        is_slow = c == SLOW_CORE
        my_cnt = jnp.where(is_slow, T_SLOW, T_FAST)
        my_base = jnp.where(is_slow, s * T_SLOW, NS * T_SLOW + s * T_FAST)
        n_chunks = my_cnt // K
        lanes = lax.iota(jnp.int32, L)
        rows_s = (rows_s0, rows_s1)
        rows_d = (rows_d0, rows_d1)
        sem_s = (sem_s0, sem_s1)
        sem_d = (sem_d0, sem_d1)

        def start_gathers(ch, b):
            pltpu.async_copy(
                h_hbm.at[idx_s.at[pl.ds(ch * K, K)]], rows_s[b], sem_s[b])
            pltpu.async_copy(
                h_hbm.at[idx_d.at[pl.ds(ch * K, K)]], rows_d[b], sem_d[b])

        def wait_gathers(ch, b):
            pltpu.make_async_copy(
                h_hbm.at[idx_s.at[pl.ds(ch * K, K)]], rows_s[b],
                sem_s[b]).wait()
            pltpu.make_async_copy(
                h_hbm.at[idx_d.at[pl.ds(ch * K, K)]], rows_d[b],
                sem_d[b]).wait()

        @pl.when(is_slow)
        def _():
            pltpu.sync_copy(src_hbm.at[pl.ds(my_base, T_SLOW)],
                            idx_s.at[pl.ds(0, T_SLOW)])
            pltpu.sync_copy(dst_hbm.at[pl.ds(my_base, T_SLOW)],
                            idx_d.at[pl.ds(0, T_SLOW)])

        @pl.when(jnp.logical_not(is_slow))
        def _():
            pltpu.sync_copy(src_hbm.at[pl.ds(my_base, T_FAST)], idx_s)
            pltpu.sync_copy(dst_hbm.at[pl.ds(my_base, T_FAST)], idx_d)

        start_gathers(0, 0)

        def compute_chunk(ch, b):
            rs, rd = rows_s[b], rows_d[b]

            def group_body(g, _):
                score = jnp.zeros((L,), jnp.float32)
                for e in range(L):
                    row = g * L + e
                    acc = rs[row, pl.ds(0, L)] * rd[row, pl.ds(0, L)]
                    for j in range(1, D_FEAT // L):
                        acc = acc + (rs[row, pl.ds(j * L, L)]
                                     * rd[row, pl.ds(j * L, L)])
                    r = acc[0]
                    for j in range(1, L):
                        r = r + acc[j]
                    score = jnp.where(lanes == e, r, score)
                out_v[pl.ds(ch * K + g * L, L)] = score
                return 0

            lax.fori_loop(0, K // L, group_body, 0, unroll=False)

        def pipe_body(cc, _):
            for b in range(2):
                ch = cc * 2 + b

                @pl.when(ch + 1 < n_chunks)
                def _():
                    start_gathers(ch + 1, 1 - b)

                wait_gathers(ch, b)
                compute_chunk(ch, b)
            return 0

        lax.fori_loop(0, n_chunks // 2, pipe_body, 0, unroll=False)

        def sig_body(v, _):
            sv = out_v[pl.ds(v * L, L)]
            out_v[pl.ds(v * L, L)] = 1.0 / (1.0 + jnp.exp(-sv))
            return 0

        lax.fori_loop(0, my_cnt // L, sig_body, 0, unroll=False)

        @pl.when(is_slow)
        def _():
            pltpu.sync_copy(out_v.at[pl.ds(0, T_SLOW)],
                            out_hbm.at[pl.ds(my_base, T_SLOW)])

        @pl.when(jnp.logical_not(is_slow))
        def _():
            pltpu.sync_copy(out_v, out_hbm.at[pl.ds(my_base, T_FAST)])

    return edge_dot


def kernel(h, edge_index):
    src = edge_index[0].astype(jnp.int32)
    dst = edge_index[1].astype(jnp.int32)
    e = src.shape[0]
    e_pad = NS * (T_SLOW + T_FAST)
    assert e <= e_pad
    pad = e_pad - e
    if pad:
        src = jnp.concatenate([src, jnp.zeros((pad,), jnp.int32)])
        dst = jnp.concatenate([dst, jnp.zeros((pad,), jnp.int32)])
    out = _build(e_pad)(h, src, dst)
    return out[:e]


# speedup vs baseline: 1.2917x; 1.2917x over previous
import functools

import jax
import jax.numpy as jnp
from jax import lax
from jax.experimental import pallas as pl
from jax.experimental.pallas import tpu as pltpu
from jax.experimental.pallas import tpu_sc as plsc

N_NODES = 10000
D_FEAT = 256
L = 16
NC, NS = 2, 16
NW = NC * NS
K = 16
NBUF = 2


@functools.cache
def _build(E_pad):
    per_w = E_pad // NW
    n_chunks = per_w // K
    assert n_chunks % NBUF == 0
    mesh = plsc.VectorSubcoreMesh(core_axis_name="c", subcore_axis_name="s")

    @functools.partial(
        pl.kernel,
        mesh=mesh,
        out_type=jax.ShapeDtypeStruct((E_pad,), jnp.float32),
        scratch_types=[
            pltpu.VMEM((per_w,), jnp.int32),
            pltpu.VMEM((per_w,), jnp.int32),
            *[pltpu.VMEM((K, D_FEAT), jnp.float32) for _ in range(2 * NBUF)],
            pltpu.VMEM((per_w,), jnp.float32),
            *[pltpu.SemaphoreType.DMA for _ in range(2 * NBUF)],
        ],
    )
    def edge_dot(h_hbm, src_hbm, dst_hbm, out_hbm, idx_s, idx_d, *rest):
        rows_s = rest[0:NBUF]
        rows_d = rest[NBUF:2 * NBUF]
        out_v = rest[2 * NBUF]
        sem_s = rest[2 * NBUF + 1:3 * NBUF + 1]
        sem_d = rest[3 * NBUF + 1:4 * NBUF + 1]
        wid = lax.axis_index("s") * NC + lax.axis_index("c")
        w_base = wid * per_w
        lanes = lax.iota(jnp.int32, L)

        def start_gathers(ch, b):
            pltpu.async_copy(
                h_hbm.at[idx_s.at[pl.ds(ch * K, K)]], rows_s[b], sem_s[b])
            pltpu.async_copy(
                h_hbm.at[idx_d.at[pl.ds(ch * K, K)]], rows_d[b], sem_d[b])

        def wait_gathers(ch, b):
            pltpu.make_async_copy(
                h_hbm.at[idx_s.at[pl.ds(ch * K, K)]], rows_s[b],
                sem_s[b]).wait()
            pltpu.make_async_copy(
                h_hbm.at[idx_d.at[pl.ds(ch * K, K)]], rows_d[b],
                sem_d[b]).wait()

        pltpu.sync_copy(src_hbm.at[pl.ds(w_base, per_w)], idx_s)
        pltpu.sync_copy(dst_hbm.at[pl.ds(w_base, per_w)], idx_d)
        for b0 in range(NBUF - 1):
            start_gathers(b0, b0)

        def compute_chunk(ch, b):
            rs, rd = rows_s[b], rows_d[b]
            score = jnp.zeros((L,), jnp.float32)
            for e in range(K):
                acc = rs[e, pl.ds(0, L)] * rd[e, pl.ds(0, L)]
                for j in range(1, D_FEAT // L):
                    acc = acc + (rs[e, pl.ds(j * L, L)]
                                 * rd[e, pl.ds(j * L, L)])
                r = acc[0]
                for j in range(1, L):
                    r = r + acc[j]
                score = jnp.where(lanes == e, r, score)
            out_v[pl.ds(ch * K, L)] = score

        def pipe_body(cc, _):
            for b in range(NBUF):
                ch = cc * NBUF + b

                @pl.when(ch + NBUF - 1 < n_chunks)
                def _():
                    start_gathers(ch + NBUF - 1, (b + NBUF - 1) % NBUF)

                wait_gathers(ch, b)
                compute_chunk(ch, b)
            return 0

        lax.fori_loop(0, n_chunks // NBUF, pipe_body, 0, unroll=False)

        def sig_body(v, _):
            sv = out_v[pl.ds(v * L, L)]
            out_v[pl.ds(v * L, L)] = 1.0 / (1.0 + jnp.exp(-sv))
            return 0

        lax.fori_loop(0, per_w // L, sig_body, 0, unroll=False)
        pltpu.sync_copy(out_v, out_hbm.at[pl.ds(w_base, per_w)])

    return edge_dot


def kernel(h, edge_index):
    src = edge_index[0].astype(jnp.int32)
    dst = edge_index[1].astype(jnp.int32)
    e = src.shape[0]
    blk = NW * K * NBUF
    e_pad = ((e + blk - 1) // blk) * blk
    pad = e_pad - e
    if pad:
        src = jnp.concatenate([src, jnp.zeros((pad,), jnp.int32)])
        dst = jnp.concatenate([dst, jnp.zeros((pad,), jnp.int32)])
    out = _build(e_pad)(h, src, dst)
    return out[:e]
